# baseline (speedup 1.0000x reference)
import functools

import jax
import jax.numpy as jnp
from jax import lax
from jax.experimental import pallas as pl
from jax.experimental.pallas import tpu as pltpu
from jax.experimental.pallas import tpu_sc as plsc

_BETA = 0.8
_LAMBDA_P = 1.0
_LAMBDA_N = 0.1
_U_LOW = 0.02
_U_UP = 0.1
_W_LOW = 0.2
_W_UP = 0.8
_K = -(_W_UP - _W_LOW) / (_U_UP - _U_LOW)
_B = _W_LOW - _K * _U_UP

_N = 16 * 512 * 512
_NC = 2
_NS = 16
_NW = _NC * _NS
_PW = _N // _NW
_CH = 16384
_NCHUNK = _PW // _CH
_L = 16
_UNROLL = 4
_NACC = 6
_PROW = _NACC * _L


def _partials_body(r_hbm, lab_hbm, u_hbm, out_hbm,
                   rb0, lb0, ub0, rb1, lb1, ub1, stage, sem0, sem1):
    wid = lax.axis_index("s") * _NC + lax.axis_index("c")
    base = wid * _PW
    bufs = ((rb0, lb0, ub0, sem0), (rb1, lb1, ub1, sem1))

    def start(c):
        rb, lb, ub, sem = bufs[c % 2]
        off = base + c * _CH
        return (pltpu.async_copy(r_hbm.at[pl.ds(off, _CH)], rb, sem),
                pltpu.async_copy(lab_hbm.at[pl.ds(off, _CH)], lb, sem),
                pltpu.async_copy(u_hbm.at[pl.ds(off, _CH)], ub, sem))

    def chunk_accum(rb, lb, ub, acc):
        def body(i, acc):
            accl = list(acc)
            o0 = i * (_L * _UNROLL)
            for uu in range(_UNROLL):
                o = o0 + uu * _L
                r = rb[pl.ds(o, _L)]
                labf = lb[pl.ds(o, _L)].astype(jnp.float32)
                u = ub[pl.ds(o, _L)]
                npos, srp, sr2p, sr, sr2, srwn = accl[uu * _NACC:(uu + 1) * _NACC]
                rp = r * labf
                rn = r - rp
                t = _K * u + _B
                w = jnp.maximum(t, _W_LOW)
                w = jnp.where(t > _W_UP, 1.0, w)
                accl[uu * _NACC:(uu + 1) * _NACC] = [
                    npos + labf,
                    srp + rp,
                    sr2p + rp * r,
                    sr + r,
                    sr2 + r * r,
                    srwn + rn * w,
                ]
            return tuple(accl)
        return lax.fori_loop(0, _CH // (_L * _UNROLL), body, acc)

    zero = jnp.zeros((_L,), jnp.float32)
    acc = (zero,) * (_NACC * _UNROLL)

    handles = start(0)
    for c in range(_NCHUNK):
        for h in handles:
            h.wait()
        if c + 1 < _NCHUNK:
            nxt = start(c + 1)
        rb, lb, ub, _ = bufs[c % 2]
        acc = chunk_accum(rb, lb, ub, acc)
        if c + 1 < _NCHUNK:
            handles = nxt

    for k in range(_NACC):
        tot = acc[k]
        for uu in range(1, _UNROLL):
            tot = tot + acc[uu * _NACC + k]
        stage[pl.ds(k * _L, _L)] = tot
    pltpu.sync_copy(stage, out_hbm.at[wid])


_partials_kernel = functools.partial(
    pl.kernel,
    out_type=jax.ShapeDtypeStruct((_NW, _PROW), jnp.float32),
    mesh=plsc.VectorSubcoreMesh(core_axis_name="c", subcore_axis_name="s"),
    scratch_types=[
        pltpu.VMEM((_CH,), jnp.float32),
        pltpu.VMEM((_CH,), jnp.int32),
        pltpu.VMEM((_CH,), jnp.float32),
        pltpu.VMEM((_CH,), jnp.float32),
        pltpu.VMEM((_CH,), jnp.int32),
        pltpu.VMEM((_CH,), jnp.float32),
        pltpu.VMEM((_PROW,), jnp.float32),
        pltpu.SemaphoreType.DMA,
        pltpu.SemaphoreType.DMA,
    ],
)(_partials_body)


def _finish_body(p_ref, o_ref):
    p = p_ref[...]
    s = [jnp.sum(p[:, k * _L:(k + 1) * _L]) for k in range(_NACC)]
    n_pos, s_rp, s_r2p, s_r, s_r2, s_rwn = s
    n_neg = _N - n_pos
    mean_p = s_rp / n_pos
    var_p = (s_r2p - s_rp * s_rp / n_pos) / (n_pos - 1.0)
    s_rn = s_r - s_rp
    s_r2n = s_r2 - s_r2p
    mean_n = s_rwn / n_neg
    var_n = (s_r2n - s_rn * s_rn / n_neg) / (n_neg - 1.0)
    loss = (jnp.maximum(_BETA - mean_p, 0.0) + _LAMBDA_N * var_p
            + mean_n + _LAMBDA_P * var_n)
    o_ref[0] = loss


_finish = pl.pallas_call(
    _finish_body,
    out_shape=jax.ShapeDtypeStruct((1,), jnp.float32),
    out_specs=pl.BlockSpec(memory_space=pltpu.SMEM),
)


def kernel(residues, pixel_level_labels, uncertainty_maps):
    r = residues.reshape(_N)
    lab = pixel_level_labels.reshape(_N).astype(jnp.int32)
    u = uncertainty_maps.reshape(_N)
    parts = _partials_kernel(r, lab, u)
    return _finish(parts)


# speedup vs baseline: 1.9431x; 1.9431x over previous
import functools

import jax
import jax.numpy as jnp
from jax import lax
from jax.experimental import pallas as pl
from jax.experimental.pallas import tpu as pltpu
from jax.experimental.pallas import tpu_sc as plsc

_BETA = 0.8
_LAMBDA_P = 1.0
_LAMBDA_N = 0.1
_U_LOW = 0.02
_U_UP = 0.1
_W_LOW = 0.2
_W_UP = 0.8
_K = -(_W_UP - _W_LOW) / (_U_UP - _U_LOW)
_B = _W_LOW - _K * _U_UP

_N = 16 * 512 * 512
_COLS = 512
_ROWS = _N // _COLS
_NC = 2
_NS = 16
_NW = _NC * _NS
_RW = _ROWS // _NW
_CR = 32
_NCHUNK = _RW // _CR
_L = 16
_NSET = 4
_NACC = 6
_PROW = _NACC * _L


def _partials_body(r_hbm, lab_hbm, u_hbm, out_hbm,
                   rb0, lb0, ub0, rb1, lb1, ub1, stage, sem0, sem1):
    wid = lax.axis_index("s") * _NC + lax.axis_index("c")
    row0 = wid * _RW
    bufs = ((rb0, lb0, ub0, sem0), (rb1, lb1, ub1, sem1))

    def start(c):
        rb, lb, ub, sem = bufs[c % 2]
        r0 = row0 + c * _CR
        return (pltpu.async_copy(r_hbm.at[pl.ds(r0, _CR), :], rb, sem),
                pltpu.async_copy(lab_hbm.at[pl.ds(r0, _CR), :], lb, sem),
                pltpu.async_copy(u_hbm.at[pl.ds(r0, _CR), :], ub, sem))

    blk_per_row = _COLS // (_L * _NSET)

    def chunk_accum(rb, lb, ub, acc):
        def body(i, acc):
            accl = list(acc)
            row = i // blk_per_row
            col0 = (i % blk_per_row) * (_L * _NSET)
            for j in range(_NSET):
                col = col0 + j * _L
                r = rb[row, pl.ds(col, _L)]
                labf = lb[row, pl.ds(col, _L)].astype(jnp.float32)
                u = ub[row, pl.ds(col, _L)]
                s = j * _NACC
                npos, srp, sr2p, sr, sr2, srwn = accl[s:s + _NACC]
                rp = r * labf
                rn = r - rp
                t = _K * u + _B
                w = jnp.maximum(t, _W_LOW)
                w = jnp.where(t > _W_UP, 1.0, w)
                accl[s:s + _NACC] = [
                    npos + labf,
                    srp + rp,
                    sr2p + rp * r,
                    sr + r,
                    sr2 + r * r,
                    srwn + rn * w,
                ]
            return tuple(accl)
        return lax.fori_loop(0, _CR * blk_per_row, body, acc)

    zero = jnp.zeros((_L,), jnp.float32)
    acc = (zero,) * (_NACC * _NSET)

    handles = start(0)
    for c in range(_NCHUNK):
        for h in handles:
            h.wait()
        if c + 1 < _NCHUNK:
            nxt = start(c + 1)
        rb, lb, ub, _ = bufs[c % 2]
        acc = chunk_accum(rb, lb, ub, acc)
        if c + 1 < _NCHUNK:
            handles = nxt

    for k in range(_NACC):
        tot = acc[k]
        for s in range(1, _NSET):
            tot = tot + acc[s * _NACC + k]
        stage[pl.ds(k * _L, _L)] = tot
    pltpu.sync_copy(stage, out_hbm.at[wid])


_partials_kernel = functools.partial(
    pl.kernel,
    out_type=jax.ShapeDtypeStruct((_NW, _PROW), jnp.float32),
    mesh=plsc.VectorSubcoreMesh(core_axis_name="c", subcore_axis_name="s"),
    scratch_types=[
        pltpu.VMEM((_CR, _COLS), jnp.float32),
        pltpu.VMEM((_CR, _COLS), jnp.int32),
        pltpu.VMEM((_CR, _COLS), jnp.float32),
        pltpu.VMEM((_CR, _COLS), jnp.float32),
        pltpu.VMEM((_CR, _COLS), jnp.int32),
        pltpu.VMEM((_CR, _COLS), jnp.float32),
        pltpu.VMEM((_PROW,), jnp.float32),
        pltpu.SemaphoreType.DMA,
        pltpu.SemaphoreType.DMA,
    ],
)(_partials_body)


def _finish_body(p_ref, o_ref):
    p = p_ref[...]
    s = [jnp.sum(p[:, k * _L:(k + 1) * _L]) for k in range(_NACC)]
    n_pos, s_rp, s_r2p, s_r, s_r2, s_rwn = s
    n_neg = _N - n_pos
    mean_p = s_rp / n_pos
    var_p = (s_r2p - s_rp * s_rp / n_pos) / (n_pos - 1.0)
    s_rn = s_r - s_rp
    s_r2n = s_r2 - s_r2p
    mean_n = s_rwn / n_neg
    var_n = (s_r2n - s_rn * s_rn / n_neg) / (n_neg - 1.0)
    loss = (jnp.maximum(_BETA - mean_p, 0.0) + _LAMBDA_N * var_p
            + mean_n + _LAMBDA_P * var_n)
    o_ref[0] = loss


_finish = pl.pallas_call(
    _finish_body,
    out_shape=jax.ShapeDtypeStruct((1,), jnp.float32),
    out_specs=pl.BlockSpec(memory_space=pltpu.SMEM),
)


def kernel(residues, pixel_level_labels, uncertainty_maps):
    r = residues.reshape(_ROWS, _COLS)
    lab = pixel_level_labels.reshape(_ROWS, _COLS).astype(jnp.int32)
    u = uncertainty_maps.reshape(_ROWS, _COLS)
    parts = _partials_kernel(r, lab, u)
    return _finish(parts)


# speedup vs baseline: 1.9442x; 1.0006x over previous
import functools

import jax
import jax.numpy as jnp
from jax import lax
from jax.experimental import pallas as pl
from jax.experimental.pallas import tpu as pltpu
from jax.experimental.pallas import tpu_sc as plsc

_BETA = 0.8
_LAMBDA_P = 1.0
_LAMBDA_N = 0.1
_U_LOW = 0.02
_U_UP = 0.1
_W_LOW = 0.2
_W_UP = 0.8
_K = -(_W_UP - _W_LOW) / (_U_UP - _U_LOW)
_B = _W_LOW - _K * _U_UP

_N = 16 * 512 * 512
_COLS = 512
_ROWS = _N // _COLS
_NC = 2
_NS = 16
_NW = _NC * _NS
_RW = _ROWS // _NW
_CR = 32
_NCHUNK = _RW // _CR
_L = 16
_NSET = 4
_NACC = 6
_PROW = _NACC * _L


def _partials_body(r_hbm, lab_hbm, u_hbm, out_hbm,
                   rb0, lb0, ub0, rb1, lb1, ub1, stage, sem0, sem1):
    wid = lax.axis_index("s") * _NC + lax.axis_index("c")
    row0 = wid * _RW
    bufs = ((rb0, lb0, ub0, sem0), (rb1, lb1, ub1, sem1))

    def start(c):
        rb, lb, ub, sem = bufs[c % 2]
        r0 = row0 + c * _CR
        return (pltpu.async_copy(r_hbm.at[pl.ds(r0, _CR), :], rb, sem),
                pltpu.async_copy(lab_hbm.at[pl.ds(r0, _CR), :], lb, sem),
                pltpu.async_copy(u_hbm.at[pl.ds(r0, _CR), :], ub, sem))

    blk_per_row = _COLS // (_L * _NSET)

    def chunk_accum(rb, lb, ub, acc):
        @plsc.parallel_loop(0, _CR * blk_per_row, unroll=2, carry=acc)
        def body(i, acc):
            accl = list(acc)
            row = i // blk_per_row
            col0 = (i % blk_per_row) * (_L * _NSET)
            for j in range(_NSET):
                col = col0 + j * _L
                r = rb[row, pl.ds(col, _L)]
                labf = lb[row, pl.ds(col, _L)].astype(jnp.float32)
                u = ub[row, pl.ds(col, _L)]
                s = j * _NACC
                npos, srp, sr2p, sr, sr2, srwn = accl[s:s + _NACC]
                rp = r * labf
                rn = r - rp
                t = _K * u + _B
                w = jnp.maximum(t, _W_LOW)
                w = jnp.where(t > _W_UP, 1.0, w)
                accl[s:s + _NACC] = [
                    npos + labf,
                    srp + rp,
                    sr2p + rp * r,
                    sr + r,
                    sr2 + r * r,
                    srwn + rn * w,
                ]
            return tuple(accl)
        return body

    zero = jnp.zeros((_L,), jnp.float32)
    acc = (zero,) * (_NACC * _NSET)

    handles = start(0)
    for c in range(_NCHUNK):
        for h in handles:
            h.wait()
        if c + 1 < _NCHUNK:
            nxt = start(c + 1)
        rb, lb, ub, _ = bufs[c % 2]
        acc = chunk_accum(rb, lb, ub, acc)
        if c + 1 < _NCHUNK:
            handles = nxt

    for k in range(_NACC):
        tot = acc[k]
        for s in range(1, _NSET):
            tot = tot + acc[s * _NACC + k]
        stage[pl.ds(k * _L, _L)] = tot
    pltpu.sync_copy(stage, out_hbm.at[wid])


_partials_kernel = functools.partial(
    pl.kernel,
    out_type=jax.ShapeDtypeStruct((_NW, _PROW), jnp.float32),
    mesh=plsc.VectorSubcoreMesh(core_axis_name="c", subcore_axis_name="s"),
    scratch_types=[
        pltpu.VMEM((_CR, _COLS), jnp.float32),
        pltpu.VMEM((_CR, _COLS), jnp.int32),
        pltpu.VMEM((_CR, _COLS), jnp.float32),
        pltpu.VMEM((_CR, _COLS), jnp.float32),
        pltpu.VMEM((_CR, _COLS), jnp.int32),
        pltpu.VMEM((_CR, _COLS), jnp.float32),
        pltpu.VMEM((_PROW,), jnp.float32),
        pltpu.SemaphoreType.DMA,
        pltpu.SemaphoreType.DMA,
    ],
)(_partials_body)


def _finish_body(p_ref, o_ref):
    p = p_ref[...]
    s = [jnp.sum(p[:, k * _L:(k + 1) * _L]) for k in range(_NACC)]
    n_pos, s_rp, s_r2p, s_r, s_r2, s_rwn = s
    n_neg = _N - n_pos
    mean_p = s_rp / n_pos
    var_p = (s_r2p - s_rp * s_rp / n_pos) / (n_pos - 1.0)
    s_rn = s_r - s_rp
    s_r2n = s_r2 - s_r2p
    mean_n = s_rwn / n_neg
    var_n = (s_r2n - s_rn * s_rn / n_neg) / (n_neg - 1.0)
    loss = (jnp.maximum(_BETA - mean_p, 0.0) + _LAMBDA_N * var_p
            + mean_n + _LAMBDA_P * var_n)
    o_ref[0] = loss


_finish = pl.pallas_call(
    _finish_body,
    out_shape=jax.ShapeDtypeStruct((1,), jnp.float32),
    out_specs=pl.BlockSpec(memory_space=pltpu.SMEM),
)


def kernel(residues, pixel_level_labels, uncertainty_maps):
    r = residues.reshape(_ROWS, _COLS)
    lab = pixel_level_labels.reshape(_ROWS, _COLS).astype(jnp.int32)
    u = uncertainty_maps.reshape(_ROWS, _COLS)
    parts = _partials_kernel(r, lab, u)
    return _finish(parts)
